# baseline (speedup 1.0000x reference)
import jax
import jax.numpy as jnp
from jax import lax
from jax.experimental import pallas as pl
from jax.experimental.pallas import tpu as pltpu
from jax.experimental.pallas import tpu_sc as plsc

B = 1024
D = 64
_NC = 2
_NS = 16
_NW = _NC * _NS
_BPW = B // _NW


_RPR = 4
_NBUF = 3
_NROUND = 2 * _BPW // _RPR


def _gather_body(u_idx, i_idx, u_tabT, i_tabT, u_out, i_out,
                 uidx_v, iidx_v, rows_v, slabs, sems, wbsem):
    wid = lax.axis_index("s") * _NC + lax.axis_index("c")
    base = wid * _BPW
    pltpu.sync_copy(u_idx.at[pl.ds(base, _BPW)], uidx_v)
    pltpu.sync_copy(i_idx.at[pl.ds(base, _BPW)], iidx_v)

    lanes = lax.iota(jnp.int32, 16)

    def u_row_scalar(jj):
        return plsc.load_gather(uidx_v, [jnp.broadcast_to(jj, (16,))])[0]

    def i_row_scalar(jj):
        return plsc.load_gather(iidx_v, [jnp.broadcast_to(jj, (16,))])[0]

    def fire(rnd, buf):
        def issue(tab, row_scalar, jb):
            for s in range(_RPR):
                r = row_scalar(jb + s)
                col0 = pl.multiple_of((r >> 7) << 7, 128)
                pltpu.async_copy(tab.at[:, pl.ds(col0, 128)],
                                 slabs.at[buf * _RPR + s], sems.at[buf])

        @pl.when(rnd < _NROUND // 2)
        def _():
            issue(u_tabT, u_row_scalar, rnd * _RPR)

        @pl.when(rnd >= _NROUND // 2)
        def _():
            issue(i_tabT, i_row_scalar, (rnd - _NROUND // 2) * _RPR)

    def drain(buf):
        for s in range(_RPR):
            pltpu.make_async_copy(u_tabT.at[:, pl.ds(0, 128)],
                                  slabs.at[buf * _RPR + s],
                                  sems.at[buf]).wait()

    def extract(rnd, buf, row_scalar, half):
        for s in range(_RPR):
            j = rnd * _RPR + s
            r = row_scalar(j - half * _BPW)
            loc = jnp.broadcast_to(r & 127, (16,))
            jv = jnp.broadcast_to(j, (16,))
            for a in range(D // 16):
                vals = plsc.load_gather(
                    slabs.at[buf * _RPR + s], [lanes + 16 * a, loc])
                plsc.store_scatter(rows_v, [jv, lanes + 16 * a], vals)

    def extract_any(rnd, buf):
        @pl.when(rnd < _NROUND // 2)
        def _():
            extract(rnd, buf, u_row_scalar, 0)

        @pl.when(rnd >= _NROUND // 2)
        def _():
            extract(rnd, buf, i_row_scalar, 1)

    fire(0, 0)
    fire(1, 1)

    def body(g, carry):
        for b in range(_NBUF):
            rnd = _NBUF * g + b

            @pl.when(rnd + 2 < _NROUND)
            def _():
                fire(rnd + 2, (b + 2) % _NBUF)

            drain(b)
            extract_any(rnd, b)

            @pl.when(rnd == _NROUND // 2 - 1)
            def _():
                pltpu.async_copy(rows_v.at[pl.ds(0, _BPW)],
                                 u_out.at[pl.ds(base, _BPW)], wbsem)
        return carry

    lax.fori_loop(0, (_NROUND - 1) // _NBUF, body, jnp.int32(0))
    tail = _NROUND - 1
    drain(tail % _NBUF)
    extract(tail, tail % _NBUF, i_row_scalar, 1)

    pltpu.make_async_copy(rows_v.at[pl.ds(0, _BPW)],
                          u_out.at[pl.ds(base, _BPW)], wbsem).wait()
    pltpu.sync_copy(rows_v.at[pl.ds(_BPW, _BPW)],
                    i_out.at[pl.ds(base, _BPW)])


def _make_gather():
    return pl.kernel(
        _gather_body,
        out_type=(
            jax.ShapeDtypeStruct((B, D), jnp.float32),
            jax.ShapeDtypeStruct((B, D), jnp.float32),
        ),
        mesh=plsc.VectorSubcoreMesh(
            core_axis_name="c", subcore_axis_name="s",
            num_cores=_NC, num_subcores=_NS,
        ),
        scratch_types=[
            pltpu.VMEM((_BPW,), jnp.int32),
            pltpu.VMEM((_BPW,), jnp.int32),
            pltpu.VMEM((2 * _BPW, D), jnp.float32),
            pltpu.VMEM((_NBUF * _RPR, D, 128), jnp.float32),
            pltpu.SemaphoreType.DMA((_NBUF,)),
            pltpu.SemaphoreType.DMA,
        ],
        compiler_params=pltpu.CompilerParams(use_tc_tiling_on_sc=True,
                                             needs_layout_passes=False),
    )


def _mlp_body(b3_ref, u_ref, v_ref, w0_ref, b0_ref, w1_ref, b1_ref,
              w2_ref, b2_ref, w3r_ref, out_ref):
    dn = (((1,), (1,)), ((), ()))
    hp = lax.Precision.DEFAULT
    u = u_ref[...]
    v = v_ref[...]
    p = u * v
    gmf_row = lax.dot_general(jnp.ones((1, D), jnp.float32), p, dn,
                              precision=hp)
    w0 = w0_ref[...]
    h = (lax.dot_general(u, w0[:, :D], dn, precision=hp)
         + lax.dot_general(v, w0[:, D:], dn, precision=hp) + b0_ref[...])
    h = jnp.maximum(h, 0.0)
    h = jnp.maximum(
        lax.dot_general(h, w1_ref[...], dn, precision=hp) + b1_ref[...], 0.0)
    h = jnp.maximum(
        lax.dot_general(h, w2_ref[...], dn, precision=hp) + b2_ref[...], 0.0)
    m_col = lax.dot_general(h, w3r_ref[...], dn, precision=hp)
    out_ref[...] = jax.nn.sigmoid(m_col + gmf_row + b3_ref[0])


def _mlp_call(u_emb, i_emb, W0, b0, W1, b1, W2, b2, W3, b3):
    w3r = W3
    return pl.pallas_call(
        _mlp_body,
        out_shape=jax.ShapeDtypeStruct((B, B), jnp.float32),
        in_specs=[pl.BlockSpec(memory_space=pltpu.SMEM)] + [
            pl.BlockSpec(memory_space=pltpu.VMEM)] * 9,
        out_specs=pl.BlockSpec(memory_space=pltpu.VMEM),
    )(b3, u_emb, i_emb, W0, b0, W1, b1, W2, b2, w3r)


def kernel(user_indices, item_indices, user_table, item_table,
           W0, b0, W1, b1, W2, b2, W3, b3):
    u_emb, i_emb = _make_gather()(user_indices.astype(jnp.int32),
                           item_indices.astype(jnp.int32),
                           user_table.T, item_table.T)
    return _mlp_call(u_emb, i_emb,
                     W0, b0.reshape(1, -1), W1, b1.reshape(1, -1),
                     W2, b2.reshape(1, -1), W3, b3)
